# baseline (device time: 51747 ns/iter reference)
import jax
import jax.numpy as jnp
from jax import lax
from jax.experimental import pallas as pl
from jax.experimental.pallas import tpu as pltpu

K = 4


def kernel(ids, E):
    v_local, d = E.shape
    t = ids.shape[0]
    th = t // 2
    ch = th // K

    my_x = lax.axis_index("x")
    my_y = lax.axis_index("y")
    ids_h = lax.dynamic_slice_in_dim(ids, my_y * th, th)
    local = ids_h - my_x * v_local
    idx = (local % v_local).astype(jnp.int32)
    in_shard = (local >= 0) & (local < v_local)
    maskf = in_shard.astype(jnp.float32)[:, None]
    mski = in_shard.astype(jnp.int32)
    counts = jnp.sum(mski.reshape(K, ch), axis=1).astype(jnp.int32)

    def body(idx_ref, mski_ref, cnt_ref, mask_ref, e_hbm, out_ref,
             gbuf, gsend, xrecv, merged, yrecv,
             gsems, xs_sems, xr_sems, ys_sems, yr_sems):
        mx = lax.axis_index("x")
        my = lax.axis_index("y")
        xpeer = (1 - mx, my)
        ypeer = (mx, 1 - my)
        my_off = my * th
        other_off = (1 - my) * th

        bar = pltpu.get_barrier_semaphore()
        for nbr in (xpeer, ypeer):
            pl.semaphore_signal(bar, inc=1, device_id=nbr,
                                device_id_type=pl.DeviceIdType.MESH)
        pl.semaphore_wait(bar, 2)

        def issue_gather(c):
            def issue(i, carry):
                @pl.when(mski_ref[i] == 1)
                def _():
                    pltpu.make_async_copy(
                        e_hbm.at[pl.ds(idx_ref[i], 1), :],
                        gbuf.at[pl.ds(i, 1), :],
                        gsems.at[c],
                    ).start()
                return carry
            lax.fori_loop(c * ch, (c + 1) * ch, issue, 0, unroll=8)

        def drain_gather(c):
            def drain(i, carry):
                pltpu.make_async_copy(
                    e_hbm.at[pl.ds(0, 1), :],
                    gbuf.at[pl.ds(0, 1), :],
                    gsems.at[c],
                ).wait()
                return carry
            lax.fori_loop(0, cnt_ref[c], drain, 0)

        def x_rdma(c):
            sl = pl.ds(c * ch, ch)
            return pltpu.make_async_remote_copy(
                src_ref=gsend.at[sl, :], dst_ref=xrecv.at[sl, :],
                send_sem=xs_sems.at[c], recv_sem=xr_sems.at[c],
                device_id=xpeer, device_id_type=pl.DeviceIdType.MESH,
            )

        def y_rdma(c):
            sl = pl.ds(c * ch, ch)
            return pltpu.make_async_remote_copy(
                src_ref=merged.at[sl, :], dst_ref=yrecv.at[sl, :],
                send_sem=ys_sems.at[c], recv_sem=yr_sems.at[c],
                device_id=ypeer, device_id_type=pl.DeviceIdType.MESH,
            )

        def reduce_and_forward(c):
            x_rdma(c).wait_recv()
            sl = pl.ds(c * ch, ch)
            merged[sl, :] = jnp.where(
                mask_ref[sl, :] != 0.0, gsend[sl, :], xrecv[sl, :]
            )
            out_ref[pl.ds(my_off + c * ch, ch), :] = (
                merged[sl, :].astype(jnp.float32)
            )
            y_rdma(c).start()

        issue_gather(0)
        for c in range(K):
            if c + 1 < K:
                issue_gather(c + 1)
            drain_gather(c)
            sl = pl.ds(c * ch, ch)
            gsend[sl, :] = gbuf[sl, :].astype(jnp.bfloat16)
            x_rdma(c).start()
            if c >= 1:
                reduce_and_forward(c - 1)
        reduce_and_forward(K - 1)

        for c in range(K):
            y_rdma(c).wait_recv()
            sl = pl.ds(c * ch, ch)
            out_ref[pl.ds(other_off + c * ch, ch), :] = (
                yrecv[sl, :].astype(jnp.float32)
            )
            x_rdma(c).wait_send()
            y_rdma(c).wait_send()

    return pl.pallas_call(
        body,
        out_shape=jax.ShapeDtypeStruct((t, d), jnp.float32),
        in_specs=[
            pl.BlockSpec(memory_space=pltpu.MemorySpace.SMEM),
            pl.BlockSpec(memory_space=pltpu.MemorySpace.SMEM),
            pl.BlockSpec(memory_space=pltpu.MemorySpace.SMEM),
            pl.BlockSpec(memory_space=pltpu.MemorySpace.VMEM),
            pl.BlockSpec(memory_space=pltpu.MemorySpace.HBM),
        ],
        out_specs=pl.BlockSpec(memory_space=pltpu.MemorySpace.VMEM),
        scratch_shapes=[
            pltpu.VMEM((th, d), jnp.float32),
            pltpu.VMEM((th, d), jnp.bfloat16),
            pltpu.VMEM((th, d), jnp.bfloat16),
            pltpu.VMEM((th, d), jnp.bfloat16),
            pltpu.VMEM((th, d), jnp.bfloat16),
            pltpu.SemaphoreType.DMA((K,)),
            pltpu.SemaphoreType.DMA((K,)),
            pltpu.SemaphoreType.DMA((K,)),
            pltpu.SemaphoreType.DMA((K,)),
            pltpu.SemaphoreType.DMA((K,)),
        ],
        compiler_params=pltpu.CompilerParams(collective_id=0),
    )(idx, mski, counts, maskf, E)


# device time: 50847 ns/iter; 1.0177x vs baseline; 1.0177x over previous
import jax
import jax.numpy as jnp
from jax import lax
from jax.experimental import pallas as pl
from jax.experimental.pallas import tpu as pltpu

K = 16


def kernel(ids, E):
    v_local, d = E.shape
    t = ids.shape[0]
    th = t // 2
    ch = th // K

    my_x = lax.axis_index("x")
    my_y = lax.axis_index("y")
    ids_h = lax.dynamic_slice_in_dim(ids, my_y * th, th)
    local = ids_h - my_x * v_local
    idx = (local % v_local).astype(jnp.int32)
    in_shard = (local >= 0) & (local < v_local)
    maskf = in_shard.astype(jnp.float32)[:, None]
    mski = in_shard.astype(jnp.int32)
    counts = jnp.sum(mski.reshape(K, ch), axis=1).astype(jnp.int32)

    def body(idx_ref, mski_ref, cnt_ref, mask_ref, e_hbm, out_ref,
             gbuf, gsend, xrecv, merged, yrecv,
             gsems, xs_sems, xr_sems, ys_sems, yr_sems):
        mx = lax.axis_index("x")
        my = lax.axis_index("y")
        xpeer = (1 - mx, my)
        ypeer = (mx, 1 - my)
        my_off = my * th
        other_off = (1 - my) * th

        bar = pltpu.get_barrier_semaphore()
        for nbr in (xpeer, ypeer):
            pl.semaphore_signal(bar, inc=1, device_id=nbr,
                                device_id_type=pl.DeviceIdType.MESH)
        pl.semaphore_wait(bar, 2)

        def issue_gather(c):
            def issue(i, carry):
                @pl.when(mski_ref[i] == 1)
                def _():
                    pltpu.make_async_copy(
                        e_hbm.at[pl.ds(idx_ref[i], 1), :],
                        gbuf.at[pl.ds(i, 1), :],
                        gsems.at[c],
                    ).start()
                return carry
            lax.fori_loop(c * ch, (c + 1) * ch, issue, 0, unroll=8)

        def drain_gather(c):
            def drain(i, carry):
                pltpu.make_async_copy(
                    e_hbm.at[pl.ds(0, 1), :],
                    gbuf.at[pl.ds(0, 1), :],
                    gsems.at[c],
                ).wait()
                return carry
            lax.fori_loop(0, cnt_ref[c], drain, 0)

        def x_rdma(c):
            sl = pl.ds(c * ch, ch)
            return pltpu.make_async_remote_copy(
                src_ref=gsend.at[sl, :], dst_ref=xrecv.at[sl, :],
                send_sem=xs_sems.at[c], recv_sem=xr_sems.at[c],
                device_id=xpeer, device_id_type=pl.DeviceIdType.MESH,
            )

        def y_rdma(c):
            sl = pl.ds(c * ch, ch)
            return pltpu.make_async_remote_copy(
                src_ref=merged.at[sl, :], dst_ref=yrecv.at[sl, :],
                send_sem=ys_sems.at[c], recv_sem=yr_sems.at[c],
                device_id=ypeer, device_id_type=pl.DeviceIdType.MESH,
            )

        def reduce_and_forward(c):
            x_rdma(c).wait_recv()
            sl = pl.ds(c * ch, ch)
            merged[sl, :] = jnp.where(
                mask_ref[sl, :] != 0.0, gsend[sl, :], xrecv[sl, :]
            )
            out_ref[pl.ds(my_off + c * ch, ch), :] = (
                merged[sl, :].astype(jnp.float32)
            )
            y_rdma(c).start()

        issue_gather(0)
        for c in range(K):
            if c + 1 < K:
                issue_gather(c + 1)
            drain_gather(c)
            sl = pl.ds(c * ch, ch)
            gsend[sl, :] = gbuf[sl, :].astype(jnp.bfloat16)
            x_rdma(c).start()
            if c >= 1:
                reduce_and_forward(c - 1)
        reduce_and_forward(K - 1)

        for c in range(K):
            y_rdma(c).wait_recv()
            sl = pl.ds(c * ch, ch)
            out_ref[pl.ds(other_off + c * ch, ch), :] = (
                yrecv[sl, :].astype(jnp.float32)
            )
            x_rdma(c).wait_send()
            y_rdma(c).wait_send()

    return pl.pallas_call(
        body,
        out_shape=jax.ShapeDtypeStruct((t, d), jnp.float32),
        in_specs=[
            pl.BlockSpec(memory_space=pltpu.MemorySpace.SMEM),
            pl.BlockSpec(memory_space=pltpu.MemorySpace.SMEM),
            pl.BlockSpec(memory_space=pltpu.MemorySpace.SMEM),
            pl.BlockSpec(memory_space=pltpu.MemorySpace.VMEM),
            pl.BlockSpec(memory_space=pltpu.MemorySpace.HBM),
        ],
        out_specs=pl.BlockSpec(memory_space=pltpu.MemorySpace.VMEM),
        scratch_shapes=[
            pltpu.VMEM((th, d), jnp.float32),
            pltpu.VMEM((th, d), jnp.bfloat16),
            pltpu.VMEM((th, d), jnp.bfloat16),
            pltpu.VMEM((th, d), jnp.bfloat16),
            pltpu.VMEM((th, d), jnp.bfloat16),
            pltpu.SemaphoreType.DMA((K,)),
            pltpu.SemaphoreType.DMA((K,)),
            pltpu.SemaphoreType.DMA((K,)),
            pltpu.SemaphoreType.DMA((K,)),
            pltpu.SemaphoreType.DMA((K,)),
        ],
        compiler_params=pltpu.CompilerParams(collective_id=0),
    )(idx, mski, counts, maskf, E)


# device time: 47381 ns/iter; 1.0921x vs baseline; 1.0732x over previous
import jax
import jax.numpy as jnp
from jax import lax
from jax.experimental import pallas as pl
from jax.experimental.pallas import tpu as pltpu

K = 8


def kernel(ids, E):
    v_local, d = E.shape
    t = ids.shape[0]
    th = t // 2
    ch = th // K

    my_x = lax.axis_index("x")
    my_y = lax.axis_index("y")
    ids_h = lax.dynamic_slice_in_dim(ids, my_y * th, th)
    local = ids_h - my_x * v_local
    idx = (local % v_local).astype(jnp.int32)
    in_shard = (local >= 0) & (local < v_local)
    maskf = in_shard.astype(jnp.float32)[:, None]
    mski = in_shard.astype(jnp.int32)
    counts = jnp.sum(mski.reshape(K, ch), axis=1).astype(jnp.int32)

    def body(idx_ref, mski_ref, cnt_ref, mask_ref, e_hbm, out_ref,
             gbuf, gsend, xrecv, merged, yrecv,
             gsems, xs_sems, xr_sems, ys_sems, yr_sems):
        mx = lax.axis_index("x")
        my = lax.axis_index("y")
        xpeer = (1 - mx, my)
        ypeer = (mx, 1 - my)
        my_off = my * th
        other_off = (1 - my) * th

        bar = pltpu.get_barrier_semaphore()
        for nbr in (xpeer, ypeer):
            pl.semaphore_signal(bar, inc=1, device_id=nbr,
                                device_id_type=pl.DeviceIdType.MESH)
        pl.semaphore_wait(bar, 2)

        def issue_gather(c):
            def issue(i, carry):
                @pl.when(mski_ref[i] == 1)
                def _():
                    pltpu.make_async_copy(
                        e_hbm.at[pl.ds(idx_ref[i], 1), :],
                        gbuf.at[pl.ds(i, 1), :],
                        gsems.at[c],
                    ).start()
                return carry
            lax.fori_loop(c * ch, (c + 1) * ch, issue, 0, unroll=8)

        def drain_gather(c):
            def drain(i, carry):
                pltpu.make_async_copy(
                    e_hbm.at[pl.ds(0, 1), :],
                    gbuf.at[pl.ds(0, 1), :],
                    gsems.at[c],
                ).wait()
                return carry
            lax.fori_loop(0, cnt_ref[c], drain, 0)

        def x_rdma(c):
            sl = pl.ds(c * ch, ch)
            return pltpu.make_async_remote_copy(
                src_ref=gsend.at[sl, :], dst_ref=xrecv.at[sl, :],
                send_sem=xs_sems.at[c], recv_sem=xr_sems.at[c],
                device_id=xpeer, device_id_type=pl.DeviceIdType.MESH,
            )

        def y_rdma(c):
            sl = pl.ds(c * ch, ch)
            return pltpu.make_async_remote_copy(
                src_ref=merged.at[sl, :], dst_ref=yrecv.at[sl, :],
                send_sem=ys_sems.at[c], recv_sem=yr_sems.at[c],
                device_id=ypeer, device_id_type=pl.DeviceIdType.MESH,
            )

        def reduce_and_forward(c):
            x_rdma(c).wait_recv()
            sl = pl.ds(c * ch, ch)
            merged[sl, :] = jnp.where(
                mask_ref[sl, :] != 0.0, gsend[sl, :], xrecv[sl, :]
            )
            out_ref[pl.ds(my_off + c * ch, ch), :] = (
                merged[sl, :].astype(jnp.float32)
            )
            y_rdma(c).start()

        issue_gather(0)
        for c in range(K):
            if c + 1 < K:
                issue_gather(c + 1)
            drain_gather(c)
            sl = pl.ds(c * ch, ch)
            gsend[sl, :] = gbuf[sl, :].astype(jnp.bfloat16)
            x_rdma(c).start()
            if c >= 1:
                reduce_and_forward(c - 1)
        reduce_and_forward(K - 1)

        for c in range(K):
            y_rdma(c).wait_recv()
            sl = pl.ds(c * ch, ch)
            out_ref[pl.ds(other_off + c * ch, ch), :] = (
                yrecv[sl, :].astype(jnp.float32)
            )
            x_rdma(c).wait_send()
            y_rdma(c).wait_send()

    return pl.pallas_call(
        body,
        out_shape=jax.ShapeDtypeStruct((t, d), jnp.float32),
        in_specs=[
            pl.BlockSpec(memory_space=pltpu.MemorySpace.SMEM),
            pl.BlockSpec(memory_space=pltpu.MemorySpace.SMEM),
            pl.BlockSpec(memory_space=pltpu.MemorySpace.SMEM),
            pl.BlockSpec(memory_space=pltpu.MemorySpace.VMEM),
            pl.BlockSpec(memory_space=pltpu.MemorySpace.HBM),
        ],
        out_specs=pl.BlockSpec(memory_space=pltpu.MemorySpace.VMEM),
        scratch_shapes=[
            pltpu.VMEM((th, d), jnp.float32),
            pltpu.VMEM((th, d), jnp.bfloat16),
            pltpu.VMEM((th, d), jnp.bfloat16),
            pltpu.VMEM((th, d), jnp.bfloat16),
            pltpu.VMEM((th, d), jnp.bfloat16),
            pltpu.SemaphoreType.DMA((K,)),
            pltpu.SemaphoreType.DMA((K,)),
            pltpu.SemaphoreType.DMA((K,)),
            pltpu.SemaphoreType.DMA((K,)),
            pltpu.SemaphoreType.DMA((K,)),
        ],
        compiler_params=pltpu.CompilerParams(collective_id=0),
    )(idx, mski, counts, maskf, E)


# device time: 47181 ns/iter; 1.0968x vs baseline; 1.0042x over previous
import jax
import jax.numpy as jnp
from jax import lax
from jax.experimental import pallas as pl
from jax.experimental.pallas import tpu as pltpu

K = 8


def kernel(ids, E):
    v_local, d = E.shape
    t = ids.shape[0]
    th = t // 2
    ch = th // K

    my_x = lax.axis_index("x")
    my_y = lax.axis_index("y")
    ids_h = lax.dynamic_slice_in_dim(ids, my_y * th, th)
    local = ids_h - my_x * v_local
    idx = (local % v_local).astype(jnp.int32)
    in_shard = (local >= 0) & (local < v_local)
    maskf = in_shard.astype(jnp.float32)[:, None]
    mski = in_shard.astype(jnp.int32)
    counts = jnp.sum(mski.reshape(K, ch), axis=1).astype(jnp.int32)

    def body(idx_ref, mski_ref, cnt_ref, mask_ref, e_hbm, out_ref,
             gbuf, gsend, xrecv, merged, yrecv,
             gsems, xs_sems, xr_sems, ys_sems, yr_sems):
        mx = lax.axis_index("x")
        my = lax.axis_index("y")
        xpeer = (1 - mx, my)
        ypeer = (mx, 1 - my)
        my_off = my * th
        other_off = (1 - my) * th

        def issue_gather(c):
            def issue(i, carry):
                @pl.when(mski_ref[i] == 1)
                def _():
                    pltpu.make_async_copy(
                        e_hbm.at[pl.ds(idx_ref[i], 1), :],
                        gbuf.at[pl.ds(i, 1), :],
                        gsems.at[c],
                    ).start()
                return carry
            lax.fori_loop(c * ch, (c + 1) * ch, issue, 0, unroll=8)

        def drain_gather(c):
            def drain(i, carry):
                pltpu.make_async_copy(
                    e_hbm.at[pl.ds(0, 1), :],
                    gbuf.at[pl.ds(0, 1), :],
                    gsems.at[c],
                ).wait()
                return carry
            lax.fori_loop(0, cnt_ref[c], drain, 0)

        def x_rdma(c):
            sl = pl.ds(c * ch, ch)
            return pltpu.make_async_remote_copy(
                src_ref=gsend.at[sl, :], dst_ref=xrecv.at[sl, :],
                send_sem=xs_sems.at[c], recv_sem=xr_sems.at[c],
                device_id=xpeer, device_id_type=pl.DeviceIdType.MESH,
            )

        def y_rdma(c):
            sl = pl.ds(c * ch, ch)
            return pltpu.make_async_remote_copy(
                src_ref=merged.at[sl, :], dst_ref=yrecv.at[sl, :],
                send_sem=ys_sems.at[c], recv_sem=yr_sems.at[c],
                device_id=ypeer, device_id_type=pl.DeviceIdType.MESH,
            )

        def reduce_and_forward(c):
            x_rdma(c).wait_recv()
            sl = pl.ds(c * ch, ch)
            merged[sl, :] = jnp.where(
                mask_ref[sl, :] != 0.0, gsend[sl, :], xrecv[sl, :]
            )
            out_ref[pl.ds(my_off + c * ch, ch), :] = (
                merged[sl, :].astype(jnp.float32)
            )
            y_rdma(c).start()

        issue_gather(0)

        bar = pltpu.get_barrier_semaphore()
        for nbr in (xpeer, ypeer):
            pl.semaphore_signal(bar, inc=1, device_id=nbr,
                                device_id_type=pl.DeviceIdType.MESH)
        pl.semaphore_wait(bar, 2)

        for c in range(K):
            if c + 1 < K:
                issue_gather(c + 1)
            drain_gather(c)
            sl = pl.ds(c * ch, ch)
            gsend[sl, :] = gbuf[sl, :].astype(jnp.bfloat16)
            x_rdma(c).start()
            if c >= 1:
                reduce_and_forward(c - 1)
        reduce_and_forward(K - 1)

        for c in range(K):
            y_rdma(c).wait_recv()
            sl = pl.ds(c * ch, ch)
            out_ref[pl.ds(other_off + c * ch, ch), :] = (
                yrecv[sl, :].astype(jnp.float32)
            )
            x_rdma(c).wait_send()
            y_rdma(c).wait_send()

    return pl.pallas_call(
        body,
        out_shape=jax.ShapeDtypeStruct((t, d), jnp.float32),
        in_specs=[
            pl.BlockSpec(memory_space=pltpu.MemorySpace.SMEM),
            pl.BlockSpec(memory_space=pltpu.MemorySpace.SMEM),
            pl.BlockSpec(memory_space=pltpu.MemorySpace.SMEM),
            pl.BlockSpec(memory_space=pltpu.MemorySpace.VMEM),
            pl.BlockSpec(memory_space=pltpu.MemorySpace.HBM),
        ],
        out_specs=pl.BlockSpec(memory_space=pltpu.MemorySpace.VMEM),
        scratch_shapes=[
            pltpu.VMEM((th, d), jnp.float32),
            pltpu.VMEM((th, d), jnp.bfloat16),
            pltpu.VMEM((th, d), jnp.bfloat16),
            pltpu.VMEM((th, d), jnp.bfloat16),
            pltpu.VMEM((th, d), jnp.bfloat16),
            pltpu.SemaphoreType.DMA((K,)),
            pltpu.SemaphoreType.DMA((K,)),
            pltpu.SemaphoreType.DMA((K,)),
            pltpu.SemaphoreType.DMA((K,)),
            pltpu.SemaphoreType.DMA((K,)),
        ],
        compiler_params=pltpu.CompilerParams(collective_id=0),
    )(idx, mski, counts, maskf, E)
